# Initial kernel scaffold; baseline (speedup 1.0000x reference)
import jax
import jax.numpy as jnp
from jax.experimental import pallas as pl


def kernel(past_lengths, past_ids, past_embeddings, ratings, pos_emb, iasig_emb, rating_emb):
    raise NotImplementedError("write your pallas kernel here")


# speedup vs baseline: 2.0614x; 2.0614x over previous
import functools

import jax
import jax.numpy as jnp
from jax.experimental import pallas as pl

B, N, D = 1024, 200, 128
_SCALE = float(D) ** 0.5


def _body(pl_ref, ids_ref, r_ref, pe_ref, posA_ref, posB_ref, t7_ref,
          l2_ref, ue_ref, m_ref):
    pe = pe_ref[...]
    r3 = r_ref[...]
    ids3 = ids_ref[...]
    posA = posA_ref[...]
    posB = posB_ref[...]
    t7 = t7_ref[...]
    bb = pe.shape[0]

    me = (ids3 != 0).astype(jnp.float32)
    mo = ((r3 != 0) & (r3 != 6)).astype(jnp.float32)

    even = (pe * _SCALE + posA[None]) * me

    oh = (r3 == jax.lax.broadcasted_iota(jnp.int32, (1, 1, 8), 2))
    oh = oh.astype(jnp.float32).reshape(bb * N, 8)
    ob = jax.lax.dot_general(oh, t7, (((1,), (0,)), ((), ())),
                             preferred_element_type=jnp.float32)
    odd = (ob.reshape(bb, N, D) + posB[None]) * mo

    ue_ref[:, :, 0:D] = even
    ue_ref[:, :, D:2 * D] = odd
    m_ref[:, :, 0:1] = me
    m_ref[:, :, 1:2] = mo
    l2_ref[...] = pl_ref[...] * 2


@functools.partial(jax.jit, static_argnames=("bb",))
def _run(past_lengths, past_ids, past_embeddings, ratings, posA, posB, t7,
         bb=8):
    grid = (B // bb,)
    l2, ue, m = pl.pallas_call(
        _body,
        grid=grid,
        in_specs=[
            pl.BlockSpec((bb, 1), lambda i: (i, 0)),
            pl.BlockSpec((bb, N, 1), lambda i: (i, 0, 0)),
            pl.BlockSpec((bb, N, 1), lambda i: (i, 0, 0)),
            pl.BlockSpec((bb, N, D), lambda i: (i, 0, 0)),
            pl.BlockSpec((N, D), lambda i: (0, 0)),
            pl.BlockSpec((N, D), lambda i: (0, 0)),
            pl.BlockSpec((8, D), lambda i: (0, 0)),
        ],
        out_specs=[
            pl.BlockSpec((bb, 1), lambda i: (i, 0)),
            pl.BlockSpec((bb, N, 2 * D), lambda i: (i, 0, 0)),
            pl.BlockSpec((bb, N, 2), lambda i: (i, 0, 0)),
        ],
        out_shape=[
            jax.ShapeDtypeStruct((B, 1), jnp.int32),
            jax.ShapeDtypeStruct((B, N, 2 * D), jnp.float32),
            jax.ShapeDtypeStruct((B, N, 2), jnp.float32),
        ],
    )(past_lengths.reshape(B, 1), past_ids.reshape(B, N, 1),
      ratings.reshape(B, N, 1), past_embeddings, posA, posB, t7)
    return l2, ue, m


def kernel(past_lengths, past_ids, past_embeddings, ratings, pos_emb,
           iasig_emb, rating_emb):
    posA = pos_emb[0::2] + iasig_emb[0]
    posB = pos_emb[1::2] + iasig_emb[1]
    t7 = jnp.concatenate([rating_emb * _SCALE,
                          jnp.zeros((1, D), jnp.float32)], axis=0)
    l2, ue, m = _run(past_lengths, past_ids, past_embeddings, ratings,
                     posA, posB, t7)
    return (l2.reshape(B), ue.reshape(B, 2 * N, D), m.reshape(B, 2 * N, 1))


# speedup vs baseline: 2.1605x; 1.0481x over previous
import functools

import jax
import jax.numpy as jnp
from jax.experimental import pallas as pl

B, N, D = 1024, 200, 128
_SCALE = float(D) ** 0.5


def _body(pl_ref, ids_ref, r_ref, pe_ref, posA_ref, posB_ref, t7_ref,
          l2_ref, ue_ref, m_ref):
    pe = pe_ref[...]
    r3 = r_ref[...]
    ids3 = ids_ref[...]
    posA = posA_ref[...]
    posB = posB_ref[...]
    t7 = t7_ref[...]
    bb = pe.shape[0]

    me = (ids3 != 0).astype(jnp.float32)
    mo = ((r3 != 0) & (r3 != 6)).astype(jnp.float32)

    even = (pe * _SCALE + posA[None]) * me

    oh = (r3 == jax.lax.broadcasted_iota(jnp.int32, (1, 1, 8), 2))
    oh = oh.astype(jnp.float32).reshape(bb * N, 8)
    ob = jax.lax.dot_general(oh, t7, (((1,), (0,)), ((), ())),
                             preferred_element_type=jnp.float32)
    odd = (ob.reshape(bb, N, D) + posB[None]) * mo

    ue_ref[:, :, 0:D] = even
    ue_ref[:, :, D:2 * D] = odd
    m_ref[:, :, 0:1] = me
    m_ref[:, :, 1:2] = mo
    l2_ref[...] = pl_ref[...] * 2


@functools.partial(jax.jit, static_argnames=("bb",))
def _run(past_lengths, past_ids, past_embeddings, ratings, posA, posB, t7,
         bb=32):
    grid = (B // bb,)
    l2, ue, m = pl.pallas_call(
        _body,
        grid=grid,
        in_specs=[
            pl.BlockSpec((bb, 1), lambda i: (i, 0)),
            pl.BlockSpec((bb, N, 1), lambda i: (i, 0, 0)),
            pl.BlockSpec((bb, N, 1), lambda i: (i, 0, 0)),
            pl.BlockSpec((bb, N, D), lambda i: (i, 0, 0)),
            pl.BlockSpec((N, D), lambda i: (0, 0)),
            pl.BlockSpec((N, D), lambda i: (0, 0)),
            pl.BlockSpec((8, D), lambda i: (0, 0)),
        ],
        out_specs=[
            pl.BlockSpec((bb, 1), lambda i: (i, 0)),
            pl.BlockSpec((bb, N, 2 * D), lambda i: (i, 0, 0)),
            pl.BlockSpec((bb, N, 2), lambda i: (i, 0, 0)),
        ],
        out_shape=[
            jax.ShapeDtypeStruct((B, 1), jnp.int32),
            jax.ShapeDtypeStruct((B, N, 2 * D), jnp.float32),
            jax.ShapeDtypeStruct((B, N, 2), jnp.float32),
        ],
    )(past_lengths.reshape(B, 1), past_ids.reshape(B, N, 1),
      ratings.reshape(B, N, 1), past_embeddings, posA, posB, t7)
    return l2, ue, m


def kernel(past_lengths, past_ids, past_embeddings, ratings, pos_emb,
           iasig_emb, rating_emb):
    posA = pos_emb[0::2] + iasig_emb[0]
    posB = pos_emb[1::2] + iasig_emb[1]
    t7 = jnp.concatenate([rating_emb * _SCALE,
                          jnp.zeros((1, D), jnp.float32)], axis=0)
    l2, ue, m = _run(past_lengths, past_ids, past_embeddings, ratings,
                     posA, posB, t7)
    return (l2.reshape(B), ue.reshape(B, 2 * N, D), m.reshape(B, 2 * N, 1))
